# Initial kernel scaffold; baseline (speedup 1.0000x reference)
import jax
import jax.numpy as jnp
from jax.experimental import pallas as pl


def kernel(x, table, bias):
    raise NotImplementedError("write your pallas kernel here")


# speedup vs baseline: 1.2181x; 1.2181x over previous
import functools

import jax
import jax.numpy as jnp
from jax import lax
from jax.experimental import pallas as pl
from jax.experimental.pallas import tpu as pltpu
from jax.experimental.pallas import tpu_sc as plsc

F = 26
FIELD = 100000
B = 16384
NC, NS, L = 2, 16, 16
NW = NC * NS
BPW = B // NW
E = BPW * F
OUTV = BPW // L

_mesh = plsc.VectorSubcoreMesh(core_axis_name="c", subcore_axis_name="s")


@functools.partial(
    pl.kernel,
    out_type=jax.ShapeDtypeStruct((B,), jnp.float32),
    mesh=_mesh,
    scratch_types=[
        pltpu.VMEM((E,), jnp.int32),
        pltpu.VMEM((E,), jnp.float32),
        pltpu.VMEM((BPW,), jnp.float32),
        pltpu.SemaphoreType.DMA,
    ],
)
def _fl_kernel(xr_hbm, table_hbm, out_hbm, idx_v, vals_v, acc_v, sem):
    wid = lax.axis_index("s") * NC + lax.axis_index("c")
    base = wid * E
    pltpu.sync_copy(xr_hbm.at[pl.ds(base, E)], idx_v)

    def _field_off(f, _):
        def _vec(j, _):
            o = f * BPW + j * L
            idx_v[pl.ds(o, L)] = idx_v[pl.ds(o, L)] + f * FIELD
            return _
        return lax.fori_loop(0, OUTV, _vec, _)

    lax.fori_loop(0, F, _field_off, 0)

    pltpu.async_copy(table_hbm.at[idx_v], vals_v, sem).wait()

    def _red(j, _):
        o = j * L
        def _acc(f, a):
            return a + vals_v[pl.ds(f * BPW + o, L)]
        acc_v[pl.ds(o, L)] = lax.fori_loop(1, F, _acc, vals_v[pl.ds(o, L)])
        return _

    lax.fori_loop(0, OUTV, _red, 0)
    pltpu.sync_copy(acc_v, out_hbm.at[pl.ds(wid * BPW, BPW)])


def kernel(x, table, bias):
    xr = x.reshape(NW, BPW, F).transpose(0, 2, 1).reshape(NW * E)
    out = _fl_kernel(xr, table.reshape(-1))
    return out.reshape(B, 1) + bias


# speedup vs baseline: 4.1885x; 3.4385x over previous
import functools

import jax
import jax.numpy as jnp
from jax import lax
from jax.experimental import pallas as pl
from jax.experimental.pallas import tpu as pltpu
from jax.experimental.pallas import tpu_sc as plsc

F = 26
FIELD = 100000
B = 16384
NC, NS, L = 2, 16, 16
NW = NC * NS
BPW = B // NW
E = BPW * F
OUTV = BPW // L
TOTAL = F * FIELD

_mesh = plsc.VectorSubcoreMesh(core_axis_name="c", subcore_axis_name="s")


@functools.partial(
    pl.kernel,
    out_type=jax.ShapeDtypeStruct((B,), jnp.float32),
    mesh=_mesh,
    scratch_types=[
        pltpu.VMEM((E,), jnp.int32),
        pltpu.VMEM((E,), jnp.float32),
        pltpu.VMEM((BPW,), jnp.float32),
        pltpu.SemaphoreType.DMA,
    ],
)
def _fl_kernel(xr_hbm, table_hbm, out_hbm, idx_v, vals_v, acc_v, sem):
    wid = lax.axis_index("s") * NC + lax.axis_index("c")
    base = wid * E
    pltpu.sync_copy(xr_hbm.at[pl.ds(base, E)], idx_v)

    def _field_off(f, _):
        def _vec(j, _):
            o = f * BPW + j * L
            idx_v[pl.ds(o, L)] = idx_v[pl.ds(o, L)] + f * FIELD
            return _
        return lax.fori_loop(0, OUTV, _vec, _)

    lax.fori_loop(0, F, _field_off, 0)

    flat = table_hbm.at[0]
    pltpu.async_copy(flat.at[idx_v], vals_v, sem).wait()

    def _red(j, _):
        o = j * L
        def _acc(f, a):
            return a + vals_v[pl.ds(f * BPW + o, L)]
        acc_v[pl.ds(o, L)] = lax.fori_loop(1, F, _acc, vals_v[pl.ds(o, L)])
        return _

    lax.fori_loop(0, OUTV, _red, 0)
    pltpu.sync_copy(acc_v, out_hbm.at[pl.ds(wid * BPW, BPW)])


def kernel(x, table, bias):
    xr = x.reshape(NW, BPW, F).transpose(0, 2, 1).reshape(NW * E)
    out = _fl_kernel(xr, table.reshape(1, TOTAL))
    return out.reshape(B, 1) + bias


# speedup vs baseline: 4.5914x; 1.0962x over previous
import functools

import jax
import jax.numpy as jnp
from jax import lax
from jax.experimental import pallas as pl
from jax.experimental.pallas import tpu as pltpu
from jax.experimental.pallas import tpu_sc as plsc

F = 26
FIELD = 100000
B = 16384
NC, NS, L = 2, 16, 16
NW = NC * NS
BPW = B // NW
E = BPW * F
OUTV = BPW // L
TOTAL = F * FIELD

_mesh = plsc.VectorSubcoreMesh(core_axis_name="c", subcore_axis_name="s")


@functools.partial(
    pl.kernel,
    out_type=jax.ShapeDtypeStruct((B,), jnp.float32),
    mesh=_mesh,
    scratch_types=[
        pltpu.VMEM((E,), jnp.int32),
        pltpu.VMEM((E,), jnp.float32),
        pltpu.VMEM((BPW,), jnp.float32),
        pltpu.SemaphoreType.DMA,
    ],
)
def _fl_kernel(xr_hbm, table_hbm, out_hbm, idx_v, vals_v, acc_v, sem):
    wid = lax.axis_index("s") * NC + lax.axis_index("c")
    base = wid * E
    pltpu.sync_copy(xr_hbm.at[pl.ds(base, E)], idx_v)

    flat = table_hbm.at[0]
    pltpu.async_copy(flat.at[idx_v], vals_v, sem).wait()

    def _red(j, _):
        o = j * L
        a = vals_v[pl.ds(o, L)]
        for f in range(1, F):
            a = a + vals_v[pl.ds(f * BPW + o, L)]
        acc_v[pl.ds(o, L)] = a
        return _

    lax.fori_loop(0, OUTV, _red, 0)
    pltpu.sync_copy(acc_v, out_hbm.at[pl.ds(wid * BPW, BPW)])


def kernel(x, table, bias):
    offs = jnp.arange(F, dtype=jnp.int32) * FIELD
    xr = (x + offs[None, :]).reshape(NW, BPW, F).transpose(0, 2, 1).reshape(NW * E)
    out = _fl_kernel(xr, table.reshape(1, TOTAL))
    return out.reshape(B, 1) + bias
